# baseline (speedup 1.0000x reference)
import functools

import jax
import jax.numpy as jnp
from jax import lax
from jax.experimental import pallas as pl
from jax.experimental.pallas import tpu as pltpu
from jax.experimental.pallas import tpu_sc as plsc

N_NODES = 10000
N_EDGES = 160000
D_FEAT = 128
D_EDGE = 16
D_HID = 256
N_CLASSES = 117

_NB = 1000
_EB = 2000


def _tca_body(x_ref, xl_ref, wav_ref, wal_ref, wm_ref, wml_ref,
              av_ref, al_ref, mv_ref, ml_ref):
    x = x_ref[...]
    xl = xl_ref[...]
    f32 = jnp.float32
    av_ref[...] = jax.lax.dot(x, wav_ref[...], preferred_element_type=f32)
    al_ref[...] = jax.lax.dot(xl, wal_ref[...], preferred_element_type=f32)
    mv_ref[...] = jax.lax.dot(x, wm_ref[...], preferred_element_type=f32)
    ml_ref[...] = jax.lax.dot(xl, wml_ref[...], preferred_element_type=f32)


def _tc_a(x, xl, w_av, w_al, wm, wml, interpret=False):
    nblk = N_NODES // _NB
    blk = lambda r: pl.BlockSpec((_NB, r), lambda i: (i, 0))
    full = lambda a, b: pl.BlockSpec((a, b), lambda i: (0, 0))
    return pl.pallas_call(
        _tca_body,
        grid=(nblk,),
        in_specs=[blk(D_FEAT), blk(D_FEAT), full(D_FEAT, 2), full(D_FEAT, 2),
                  full(D_FEAT, D_FEAT), full(D_FEAT, D_FEAT)],
        out_specs=[blk(2), blk(2), blk(D_FEAT), blk(D_FEAT)],
        out_shape=[jax.ShapeDtypeStruct((N_NODES, 2), jnp.float32),
                   jax.ShapeDtypeStruct((N_NODES, 2), jnp.float32),
                   jax.ShapeDtypeStruct((N_NODES, D_FEAT), jnp.float32),
                   jax.ShapeDtypeStruct((N_NODES, D_FEAT), jnp.float32)],
        interpret=interpret,
    )(x, xl, w_av, w_al, wm, wml)


def _tcb_body(x_ref, xl_ref, aggv_ref, aggl_ref, w1d_ref, w1s_ref,
              pd_ref, ps_ref):
    nv = jnp.maximum(x_ref[...] + aggv_ref[...], 0.0)
    nl = jnp.maximum(xl_ref[...] + aggl_ref[...], 0.0)
    cat = jnp.concatenate([nv, nl], axis=1)
    f32 = jnp.float32
    pd_ref[...] = jax.lax.dot(
        cat, w1d_ref[...], preferred_element_type=f32).astype(jnp.bfloat16)
    ps_ref[...] = jax.lax.dot(
        cat, w1s_ref[...], preferred_element_type=f32).astype(jnp.bfloat16)


def _tc_b(x, xl, aggv, aggl, w1d, w1s, interpret=False):
    nblk = N_NODES // _NB
    blk = lambda r: pl.BlockSpec((_NB, r), lambda i: (i, 0))
    full = lambda a, b: pl.BlockSpec((a, b), lambda i: (0, 0))
    return pl.pallas_call(
        _tcb_body,
        grid=(nblk,),
        in_specs=[blk(D_FEAT), blk(D_FEAT), blk(D_FEAT), blk(D_FEAT),
                  full(2 * D_FEAT, D_HID), full(2 * D_FEAT, D_HID)],
        out_specs=[blk(D_HID), blk(D_HID)],
        out_shape=[jax.ShapeDtypeStruct((N_NODES, D_HID), jnp.bfloat16),
                   jax.ShapeDtypeStruct((N_NODES, D_HID), jnp.bfloat16)],
        interpret=interpret,
    )(x, xl, aggv, aggl, w1d, w1s)


def _tcc_body(bd_ref, bs_ref, ea_ref, w1e_ref, b1_ref, w2_ref, b2_ref,
              out_ref):
    f32 = jnp.float32
    e = jax.lax.dot(ea_ref[...], w1e_ref[...], preferred_element_type=f32)
    base = bd_ref[...].astype(f32) + bs_ref[...].astype(f32)
    h = jnp.maximum(base + e + b1_ref[...], 0.0)
    logit = jax.lax.dot(h, w2_ref[...], preferred_element_type=f32) + b2_ref[...]
    out_ref[...] = jax.nn.sigmoid(logit)


def _tc_c(bd, bs, ea, w1e, b1, w2, b2, interpret=False):
    nblk = N_EDGES // _EB
    blk = lambda r: pl.BlockSpec((_EB, r), lambda i: (i, 0))
    full = lambda a, b: pl.BlockSpec((a, b), lambda i: (0, 0))
    return pl.pallas_call(
        _tcc_body,
        grid=(nblk,),
        in_specs=[blk(D_HID), blk(D_HID), blk(D_EDGE),
                  full(D_EDGE, D_HID), full(1, D_HID),
                  full(D_HID, N_CLASSES), full(1, N_CLASSES)],
        out_specs=blk(N_CLASSES),
        out_shape=jax.ShapeDtypeStruct((N_EDGES, N_CLASSES), jnp.float32),
        interpret=interpret,
    )(bd, bs, ea, w1e, b1, w2, b2)


_SC1_EPW = N_EDGES // 16
_SC1_K = 80
_SROWS = 80
_NPAD = 10048
_NPW = 632


def _sc1_body(src_hbm, dst_hbm, av_hbm, al_hbm, mv_hbm, ml_hbm,
              aggv_hbm, aggl_hbm,
              a_tab, s_part, sc1_c, dc1_c, idx_g, idx_c, rows_v, sh_all, sem):
    c = lax.axis_index("c")
    t = lax.axis_index("s")
    ebase = t * _SC1_EPW
    iota16 = lax.iota(jnp.int32, 16)

    if True:
        def zs(i, _):
            for cc in range(8):
                s_part[i, pl.ds(cc * 16, 16)] = jnp.zeros((16,), jnp.float32)
                rows_v[i, pl.ds(cc * 16, 16)] = jnp.zeros((16,), jnp.float32)
            return ()
        lax.fori_loop(0, _SROWS, zs, ())

        for i in range(5):
            idx_g[pl.ds(i * 16, 16)] = iota16 + (_NPAD + i * 16)

        @pl.when(t == 0)
        def _():
            pltpu.sync_copy(rows_v, sh_all.at[pl.ds(_NPAD, _SROWS)])
        zbase = pl.multiple_of(t * _NPW, 8)
        for q in range(7):
            pltpu.sync_copy(rows_v, sh_all.at[pl.ds(zbase + q * 80, 80)])

        @pl.when(t < 15)
        def _():
            pltpu.sync_copy(rows_v.at[pl.ds(0, 72)],
                            sh_all.at[pl.ds(zbase + 560, 72)])

        @pl.when(t == 15)
        def _():
            pltpu.sync_copy(rows_v.at[pl.ds(0, 8)],
                            sh_all.at[pl.ds(15 * _NPW + 560, 8)])

        @pl.when(c == 0)
        def _():
            pltpu.sync_copy(av_hbm, a_tab)

        @pl.when(c == 1)
        def _():
            pltpu.sync_copy(al_hbm, a_tab)

        plsc.subcore_barrier()

        def p1c(k, _):
            pltpu.sync_copy(src_hbm.at[pl.ds(ebase + k * 2000, 2000)], sc1_c)
            pltpu.sync_copy(dst_hbm.at[pl.ds(ebase + k * 2000, 2000)], dc1_c)

            def p1(i, _):
                sv = sc1_c[pl.ds(i * 16, 16)]
                dv = dc1_c[pl.ds(i * 16, 16)]
                asg = plsc.load_gather(a_tab, [sv * 2])
                adg = plsc.load_gather(a_tab, [dv * 2 + 1])
                l = asg + adg
                l = jnp.where(l >= 0.0, l, l * 0.2)
                e = jnp.exp(l)
                plsc.addupdate_scatter(
                    s_part, [lax.shift_right_logical(dv, 7), dv & 127], e)
                return ()
            lax.fori_loop(0, 125, p1, ())
            return ()
        lax.fori_loop(0, _SC1_EPW // 2000, p1c, ())

        pltpu.sync_copy(s_part, sh_all.at[idx_g], add=True)
        plsc.subcore_barrier()
        pltpu.sync_copy(sh_all.at[pl.ds(_NPAD, _SROWS)], s_part)

        def p2(j, _):
            pltpu.sync_copy(src_hbm.at[pl.ds(ebase + j * _SC1_K, _SC1_K)], idx_g)
            pltpu.sync_copy(dst_hbm.at[pl.ds(ebase + j * _SC1_K, _SC1_K)], idx_c)

            @pl.when(c == 0)
            def _():
                pltpu.async_copy(mv_hbm.at[idx_g], rows_v, sem).wait()

            @pl.when(c == 1)
            def _():
                pltpu.async_copy(ml_hbm.at[idx_g], rows_v, sem).wait()

            def scale_group(g, _):
                sv = idx_g[pl.ds(g * 16, 16)]
                dv = idx_c[pl.ds(g * 16, 16)]
                asg = plsc.load_gather(a_tab, [sv * 2])
                adg = plsc.load_gather(a_tab, [dv * 2 + 1])
                l = asg + adg
                l = jnp.where(l >= 0.0, l, l * 0.2)
                e = jnp.exp(l)
                sg = plsc.load_gather(
                    s_part, [lax.shift_right_logical(dv, 7), dv & 127])
                scale = e / (sg + 1e-16)
                for r in range(16):
                    sc = scale[r]
                    row = g * 16 + r
                    for cc in range(8):
                        rows_v[row, pl.ds(cc * 16, 16)] = (
                            rows_v[row, pl.ds(cc * 16, 16)] * sc)
                return ()
            lax.fori_loop(0, _SC1_K // 16, scale_group, ())

            pltpu.sync_copy(rows_v, sh_all.at[idx_c], add=True)
            return ()
        lax.fori_loop(0, _SC1_EPW // _SC1_K, p2, ())

        plsc.subcore_barrier()

        @pl.when((c == 0) & (t < 15))
        def _():
            pltpu.sync_copy(sh_all.at[pl.ds(zbase, _NPW)],
                            aggv_hbm.at[pl.ds(zbase, _NPW)])

        @pl.when((c == 0) & (t == 15))
        def _():
            pltpu.sync_copy(sh_all.at[pl.ds(15 * _NPW, _NPAD - 15 * _NPW)],
                            aggv_hbm.at[pl.ds(15 * _NPW, _NPAD - 15 * _NPW)])

        @pl.when((c == 1) & (t < 15))
        def _():
            pltpu.sync_copy(sh_all.at[pl.ds(zbase, _NPW)],
                            aggl_hbm.at[pl.ds(zbase, _NPW)])

        @pl.when((c == 1) & (t == 15))
        def _():
            pltpu.sync_copy(sh_all.at[pl.ds(15 * _NPW, _NPAD - 15 * _NPW)],
                            aggl_hbm.at[pl.ds(15 * _NPW, _NPAD - 15 * _NPW)])


def _sc_1(src, dst, av_flat, al_flat, mv, ml):
    f = pl.kernel(
        _sc1_body,
        mesh=plsc.VectorSubcoreMesh(core_axis_name="c", subcore_axis_name="s"),
        compiler_params=pltpu.CompilerParams(needs_layout_passes=False),
        out_type=[jax.ShapeDtypeStruct((_NPAD, D_FEAT), jnp.float32),
                  jax.ShapeDtypeStruct((_NPAD, D_FEAT), jnp.float32)],
        scratch_types=[pltpu.VMEM((2 * N_NODES,), jnp.float32),
                       pltpu.VMEM((_SROWS, 128), jnp.float32),
                       pltpu.VMEM((2000,), jnp.int32),
                       pltpu.VMEM((2000,), jnp.int32),
                       pltpu.VMEM((_SC1_K,), jnp.int32),
                       pltpu.VMEM((_SC1_K,), jnp.int32),
                       pltpu.VMEM((_SC1_K, D_FEAT), jnp.float32),
                       pltpu.VMEM_SHARED((_NPAD + _SROWS, D_FEAT), jnp.float32),
                       pltpu.SemaphoreType.DMA],
    )
    return f(src, dst, av_flat, al_flat, mv, ml)


_SC2_CHUNK = 40
_SC2_EPW = N_EDGES // 32


def _sc2_body(src_hbm, dst_hbm, pd_hbm, ps_hbm, outd_hbm, outs_hbm,
              idx_v, rows_v, sem):
    c = lax.axis_index("c")
    s = lax.axis_index("s")
    wid = s * 2 + c
    base = wid * _SC2_EPW
    k = _SC2_CHUNK

    def chunk(j, _):
        off = base + j * k
        pltpu.sync_copy(dst_hbm.at[pl.ds(off, k)], idx_v)
        pltpu.async_copy(pd_hbm.at[idx_v], rows_v, sem).wait()
        pltpu.sync_copy(rows_v, outd_hbm.at[pl.ds(off, k)])
        pltpu.sync_copy(src_hbm.at[pl.ds(off, k)], idx_v)
        pltpu.async_copy(ps_hbm.at[idx_v], rows_v, sem).wait()
        pltpu.sync_copy(rows_v, outs_hbm.at[pl.ds(off, k)])
        return ()

    lax.fori_loop(0, _SC2_EPW // k, chunk, ())


def _sc_2(src, dst, pd, ps):
    f = pl.kernel(
        _sc2_body,
        mesh=plsc.VectorSubcoreMesh(core_axis_name="c", subcore_axis_name="s"),
        compiler_params=pltpu.CompilerParams(needs_layout_passes=False),
        out_type=[jax.ShapeDtypeStruct((N_EDGES, D_HID // 2), jnp.int32),
                  jax.ShapeDtypeStruct((N_EDGES, D_HID // 2), jnp.int32)],
        scratch_types=[pltpu.VMEM((_SC2_CHUNK,), jnp.int32),
                       pltpu.VMEM((_SC2_CHUNK, D_HID // 2), jnp.int32),
                       pltpu.SemaphoreType.DMA],
    )
    pd_i = lax.bitcast_convert_type(
        pd.reshape(N_NODES, D_HID // 2, 2), jnp.int32)
    ps_i = lax.bitcast_convert_type(
        ps.reshape(N_NODES, D_HID // 2, 2), jnp.int32)
    return f(src, dst, pd_i, ps_i)


def _impl(x, x_lang, edge_index, edge_attr, W_att, W_msg, W_att_lang,
          W_msg_lang, W1, b1, W2, b2, interpret=False):
    src = edge_index[0].astype(jnp.int32)
    dst = edge_index[1].astype(jnp.int32)
    w_av = jnp.stack([W_att[:D_FEAT, 0], W_att[D_FEAT:, 0]], axis=1)
    w_al = jnp.stack([W_att_lang[:D_FEAT, 0], W_att_lang[D_FEAT:, 0]], axis=1)
    w1d = W1[0:2 * D_FEAT]
    w1s = jnp.concatenate([W1[2 * D_FEAT + D_EDGE + D_FEAT:],
                           W1[2 * D_FEAT + D_EDGE:2 * D_FEAT + D_EDGE + D_FEAT]],
                          axis=0)
    w1e = W1[2 * D_FEAT:2 * D_FEAT + D_EDGE]

    av, al, mv, ml = _tc_a(x, x_lang, w_av, w_al, W_msg, W_msg_lang,
                           interpret=interpret)

    if interpret:
        def _agg(a2, m):
            a_s, a_d = a2[:, 0], a2[:, 1]
            l = a_s[src] + a_d[dst]
            l = jnp.where(l >= 0, l, 0.2 * l)
            e = jnp.exp(l)
            s = jax.ops.segment_sum(e, dst, num_segments=N_NODES)
            scale = e / (s[dst] + 1e-16)
            return jax.ops.segment_sum(m[src] * scale[:, None], dst,
                                       num_segments=N_NODES)
        aggv = _agg(av, mv)
        aggl = _agg(al, ml)
    else:
        aggv, aggl = _sc_1(src, dst, av.reshape(2 * N_NODES),
                           al.reshape(2 * N_NODES), mv, ml)
        aggv = aggv[:N_NODES]
        aggl = aggl[:N_NODES]

    pd, ps = _tc_b(x, x_lang, aggv, aggl, w1d, w1s, interpret=interpret)

    if interpret:
        bd = pd[dst]
        bs = ps[src]
    else:
        bd_i, bs_i = _sc_2(src, dst, pd, ps)
        bd = lax.bitcast_convert_type(bd_i, jnp.bfloat16).reshape(
            N_EDGES, D_HID)
        bs = lax.bitcast_convert_type(bs_i, jnp.bfloat16).reshape(
            N_EDGES, D_HID)

    return _tc_c(bd, bs, edge_attr, w1e, b1.reshape(1, D_HID), W2,
                 b2.reshape(1, N_CLASSES), interpret=interpret)


def kernel(x, x_lang, edge_index, edge_attr, W_att, W_msg, W_att_lang,
           W_msg_lang, W1, b1, W2, b2):
    return _impl(x, x_lang, edge_index, edge_attr, W_att, W_msg, W_att_lang,
                 W_msg_lang, W1, b1, W2, b2, interpret=False)


# speedup vs baseline: 2.3951x; 2.3951x over previous
import functools

import jax
import jax.numpy as jnp
from jax import lax
from jax.experimental import pallas as pl
from jax.experimental.pallas import tpu as pltpu
from jax.experimental.pallas import tpu_sc as plsc

N_NODES = 10000
N_EDGES = 160000
D_FEAT = 128
D_EDGE = 16
D_HID = 256
N_CLASSES = 117

_NB = 1000
_EB = 2000


def _tca_body(x_ref, xl_ref, wav_ref, wal_ref, wm_ref, wml_ref,
              av_ref, al_ref, mv_ref, ml_ref):
    x = x_ref[...]
    xl = xl_ref[...]
    f32 = jnp.float32
    av_ref[...] = jax.lax.dot(x, wav_ref[...], preferred_element_type=f32)
    al_ref[...] = jax.lax.dot(xl, wal_ref[...], preferred_element_type=f32)
    mv_ref[...] = jax.lax.dot(x, wm_ref[...], preferred_element_type=f32)
    ml_ref[...] = jax.lax.dot(xl, wml_ref[...], preferred_element_type=f32)


def _tc_a(x, xl, w_av, w_al, wm, wml, interpret=False):
    nblk = N_NODES // _NB
    blk = lambda r: pl.BlockSpec((_NB, r), lambda i: (i, 0))
    full = lambda a, b: pl.BlockSpec((a, b), lambda i: (0, 0))
    return pl.pallas_call(
        _tca_body,
        grid=(nblk,),
        in_specs=[blk(D_FEAT), blk(D_FEAT), full(D_FEAT, 2), full(D_FEAT, 2),
                  full(D_FEAT, D_FEAT), full(D_FEAT, D_FEAT)],
        out_specs=[blk(2), blk(2), blk(D_FEAT), blk(D_FEAT)],
        out_shape=[jax.ShapeDtypeStruct((N_NODES, 2), jnp.float32),
                   jax.ShapeDtypeStruct((N_NODES, 2), jnp.float32),
                   jax.ShapeDtypeStruct((N_NODES, D_FEAT), jnp.float32),
                   jax.ShapeDtypeStruct((N_NODES, D_FEAT), jnp.float32)],
        interpret=interpret,
    )(x, xl, w_av, w_al, wm, wml)


def _tcb_body(x_ref, xl_ref, aggv_ref, aggl_ref, w1d_ref, w1s_ref,
              pd_ref, ps_ref):
    nv = jnp.maximum(x_ref[...] + aggv_ref[...], 0.0)
    nl = jnp.maximum(xl_ref[...] + aggl_ref[...], 0.0)
    cat = jnp.concatenate([nv, nl], axis=1)
    f32 = jnp.float32

    def pack(m):
        lo = lax.bitcast_convert_type(m[:, :D_FEAT], jnp.int32)
        hi = lax.bitcast_convert_type(m[:, D_FEAT:], jnp.int32)
        lo = lax.shift_right_logical(lo + 0x8000, 16)
        hi = (hi + 0x8000) & jnp.int32(-65536)
        return lo | hi

    pd_ref[...] = pack(jax.lax.dot(cat, w1d_ref[...],
                                   preferred_element_type=f32))
    ps_ref[...] = pack(jax.lax.dot(cat, w1s_ref[...],
                                   preferred_element_type=f32))


def _tc_b(x, xl, aggv, aggl, w1d, w1s, interpret=False):
    nblk = N_NODES // _NB
    blk = lambda r: pl.BlockSpec((_NB, r), lambda i: (i, 0))
    full = lambda a, b: pl.BlockSpec((a, b), lambda i: (0, 0))
    return pl.pallas_call(
        _tcb_body,
        grid=(nblk,),
        in_specs=[blk(D_FEAT), blk(D_FEAT), blk(D_FEAT), blk(D_FEAT),
                  full(2 * D_FEAT, D_HID), full(2 * D_FEAT, D_HID)],
        out_specs=[blk(D_HID // 2), blk(D_HID // 2)],
        out_shape=[jax.ShapeDtypeStruct((N_NODES, D_HID // 2), jnp.int32),
                   jax.ShapeDtypeStruct((N_NODES, D_HID // 2), jnp.int32)],
        interpret=interpret,
    )(x, xl, aggv, aggl, w1d, w1s)


def _tcc_body(bd_ref, bs_ref, ea_ref, w1e_ref, b1_ref, w2_ref, b2_ref,
              out_ref):
    f32 = jnp.float32
    e = jax.lax.dot(ea_ref[...], w1e_ref[...], preferred_element_type=f32)

    def unpack(w):
        lo = lax.bitcast_convert_type(lax.shift_left(w, 16), f32)
        hi = lax.bitcast_convert_type(w & jnp.int32(-65536), f32)
        return lo, hi

    dlo, dhi = unpack(bd_ref[...])
    slo, shi = unpack(bs_ref[...])
    base = jnp.concatenate([dlo + slo, dhi + shi], axis=1)
    h = jnp.maximum(base + e + b1_ref[...], 0.0)
    logit = jax.lax.dot(h, w2_ref[...], preferred_element_type=f32) + b2_ref[...]
    out_ref[...] = jax.nn.sigmoid(logit)


def _tc_c(bd, bs, ea, w1e, b1, w2, b2, interpret=False):
    nblk = N_EDGES // _EB
    blk = lambda r: pl.BlockSpec((_EB, r), lambda i: (i, 0))
    full = lambda a, b: pl.BlockSpec((a, b), lambda i: (0, 0))
    return pl.pallas_call(
        _tcc_body,
        grid=(nblk,),
        in_specs=[blk(D_HID // 2), blk(D_HID // 2), blk(D_EDGE),
                  full(D_EDGE, D_HID), full(1, D_HID),
                  full(D_HID, N_CLASSES), full(1, N_CLASSES)],
        out_specs=blk(N_CLASSES),
        out_shape=jax.ShapeDtypeStruct((N_EDGES, N_CLASSES), jnp.float32),
        interpret=interpret,
    )(bd, bs, ea, w1e, b1, w2, b2)


_SC1_EPW = N_EDGES // 16
_SC1_K = 80
_SROWS = 80
_NPAD = 10048
_NPW = 632


def _sc1_body(src_hbm, dst_hbm, av_hbm, al_hbm, mv_hbm, ml_hbm,
              aggv_hbm, aggl_hbm,
              a_tab, s_part, sc1_c, dc1_c, idx_g, idx_c, rows_v, sh_all, sem):
    c = lax.axis_index("c")
    t = lax.axis_index("s")
    ebase = t * _SC1_EPW
    iota16 = lax.iota(jnp.int32, 16)

    if True:
        def zs(i, _):
            for cc in range(8):
                s_part[i, pl.ds(cc * 16, 16)] = jnp.zeros((16,), jnp.float32)
                rows_v[i, pl.ds(cc * 16, 16)] = jnp.zeros((16,), jnp.float32)
            return ()
        lax.fori_loop(0, _SROWS, zs, ())

        for i in range(5):
            idx_g[pl.ds(i * 16, 16)] = iota16 + (_NPAD + i * 16)

        @pl.when(t == 0)
        def _():
            pltpu.sync_copy(rows_v, sh_all.at[pl.ds(_NPAD, _SROWS)])
        zbase = pl.multiple_of(t * _NPW, 8)
        for q in range(7):
            pltpu.sync_copy(rows_v, sh_all.at[pl.ds(zbase + q * 80, 80)])

        @pl.when(t < 15)
        def _():
            pltpu.sync_copy(rows_v.at[pl.ds(0, 72)],
                            sh_all.at[pl.ds(zbase + 560, 72)])

        @pl.when(t == 15)
        def _():
            pltpu.sync_copy(rows_v.at[pl.ds(0, 8)],
                            sh_all.at[pl.ds(15 * _NPW + 560, 8)])

        @pl.when(c == 0)
        def _():
            pltpu.sync_copy(av_hbm, a_tab)

        @pl.when(c == 1)
        def _():
            pltpu.sync_copy(al_hbm, a_tab)

        plsc.subcore_barrier()

        def p1c(k, _):
            pltpu.sync_copy(src_hbm.at[pl.ds(ebase + k * 2000, 2000)], sc1_c)
            pltpu.sync_copy(dst_hbm.at[pl.ds(ebase + k * 2000, 2000)], dc1_c)

            def p1(i, _):
                sv = sc1_c[pl.ds(i * 16, 16)]
                dv = dc1_c[pl.ds(i * 16, 16)]
                asg = plsc.load_gather(a_tab, [sv * 2])
                adg = plsc.load_gather(a_tab, [dv * 2 + 1])
                l = asg + adg
                l = jnp.where(l >= 0.0, l, l * 0.2)
                e = jnp.exp(l)
                plsc.addupdate_scatter(
                    s_part, [lax.shift_right_logical(dv, 7), dv & 127], e)
                return ()
            lax.fori_loop(0, 125, p1, ())
            return ()
        lax.fori_loop(0, _SC1_EPW // 2000, p1c, ())

        pltpu.sync_copy(s_part, sh_all.at[idx_g], add=True)
        plsc.subcore_barrier()
        pltpu.sync_copy(sh_all.at[pl.ds(_NPAD, _SROWS)], s_part)

        def p2(j, _):
            pltpu.sync_copy(src_hbm.at[pl.ds(ebase + j * _SC1_K, _SC1_K)], idx_g)
            pltpu.sync_copy(dst_hbm.at[pl.ds(ebase + j * _SC1_K, _SC1_K)], idx_c)

            @pl.when(c == 0)
            def _():
                pltpu.async_copy(mv_hbm.at[idx_g], rows_v, sem).wait()

            @pl.when(c == 1)
            def _():
                pltpu.async_copy(ml_hbm.at[idx_g], rows_v, sem).wait()

            def scale_group(g, _):
                sv = idx_g[pl.ds(g * 16, 16)]
                dv = idx_c[pl.ds(g * 16, 16)]
                asg = plsc.load_gather(a_tab, [sv * 2])
                adg = plsc.load_gather(a_tab, [dv * 2 + 1])
                l = asg + adg
                l = jnp.where(l >= 0.0, l, l * 0.2)
                e = jnp.exp(l)
                sg = plsc.load_gather(
                    s_part, [lax.shift_right_logical(dv, 7), dv & 127])
                scale = e / (sg + 1e-16)
                for r in range(16):
                    sc = scale[r]
                    row = g * 16 + r
                    for cc in range(8):
                        rows_v[row, pl.ds(cc * 16, 16)] = (
                            rows_v[row, pl.ds(cc * 16, 16)] * sc)
                return ()
            lax.fori_loop(0, _SC1_K // 16, scale_group, ())

            pltpu.sync_copy(rows_v, sh_all.at[idx_c], add=True)
            return ()
        lax.fori_loop(0, _SC1_EPW // _SC1_K, p2, ())

        plsc.subcore_barrier()

        @pl.when((c == 0) & (t < 15))
        def _():
            pltpu.sync_copy(sh_all.at[pl.ds(zbase, _NPW)],
                            aggv_hbm.at[pl.ds(zbase, _NPW)])

        @pl.when((c == 0) & (t == 15))
        def _():
            pltpu.sync_copy(sh_all.at[pl.ds(15 * _NPW, _NPAD - 15 * _NPW)],
                            aggv_hbm.at[pl.ds(15 * _NPW, _NPAD - 15 * _NPW)])

        @pl.when((c == 1) & (t < 15))
        def _():
            pltpu.sync_copy(sh_all.at[pl.ds(zbase, _NPW)],
                            aggl_hbm.at[pl.ds(zbase, _NPW)])

        @pl.when((c == 1) & (t == 15))
        def _():
            pltpu.sync_copy(sh_all.at[pl.ds(15 * _NPW, _NPAD - 15 * _NPW)],
                            aggl_hbm.at[pl.ds(15 * _NPW, _NPAD - 15 * _NPW)])


def _sc_1(src, dst, av_flat, al_flat, mv, ml):
    f = pl.kernel(
        _sc1_body,
        mesh=plsc.VectorSubcoreMesh(core_axis_name="c", subcore_axis_name="s"),
        compiler_params=pltpu.CompilerParams(needs_layout_passes=False),
        out_type=[jax.ShapeDtypeStruct((_NPAD, D_FEAT), jnp.float32),
                  jax.ShapeDtypeStruct((_NPAD, D_FEAT), jnp.float32)],
        scratch_types=[pltpu.VMEM((2 * N_NODES,), jnp.float32),
                       pltpu.VMEM((_SROWS, 128), jnp.float32),
                       pltpu.VMEM((2000,), jnp.int32),
                       pltpu.VMEM((2000,), jnp.int32),
                       pltpu.VMEM((_SC1_K,), jnp.int32),
                       pltpu.VMEM((_SC1_K,), jnp.int32),
                       pltpu.VMEM((_SC1_K, D_FEAT), jnp.float32),
                       pltpu.VMEM_SHARED((_NPAD + _SROWS, D_FEAT), jnp.float32),
                       pltpu.SemaphoreType.DMA],
    )
    return f(src, dst, av_flat, al_flat, mv, ml)


_SC2_CHUNK = 40
_SC2_EPW = N_EDGES // 32


def _sc2_body(src_hbm, dst_hbm, pd_hbm, ps_hbm, outd_hbm, outs_hbm,
              idx_v, rows_v, sem):
    c = lax.axis_index("c")
    s = lax.axis_index("s")
    wid = s * 2 + c
    base = wid * _SC2_EPW
    k = _SC2_CHUNK

    def chunk(j, _):
        off = base + j * k
        pltpu.sync_copy(dst_hbm.at[pl.ds(off, k)], idx_v)
        pltpu.async_copy(pd_hbm.at[idx_v], rows_v, sem).wait()
        pltpu.sync_copy(rows_v, outd_hbm.at[pl.ds(off, k)])
        pltpu.sync_copy(src_hbm.at[pl.ds(off, k)], idx_v)
        pltpu.async_copy(ps_hbm.at[idx_v], rows_v, sem).wait()
        pltpu.sync_copy(rows_v, outs_hbm.at[pl.ds(off, k)])
        return ()

    lax.fori_loop(0, _SC2_EPW // k, chunk, ())


def _sc_2(src, dst, pd, ps):
    f = pl.kernel(
        _sc2_body,
        mesh=plsc.VectorSubcoreMesh(core_axis_name="c", subcore_axis_name="s"),
        compiler_params=pltpu.CompilerParams(needs_layout_passes=False),
        out_type=[jax.ShapeDtypeStruct((N_EDGES, D_HID // 2), jnp.int32),
                  jax.ShapeDtypeStruct((N_EDGES, D_HID // 2), jnp.int32)],
        scratch_types=[pltpu.VMEM((_SC2_CHUNK,), jnp.int32),
                       pltpu.VMEM((_SC2_CHUNK, D_HID // 2), jnp.int32),
                       pltpu.SemaphoreType.DMA],
    )
    return f(src, dst, pd, ps)


def _impl(x, x_lang, edge_index, edge_attr, W_att, W_msg, W_att_lang,
          W_msg_lang, W1, b1, W2, b2, interpret=False):
    src = edge_index[0].astype(jnp.int32)
    dst = edge_index[1].astype(jnp.int32)
    w_av = jnp.stack([W_att[:D_FEAT, 0], W_att[D_FEAT:, 0]], axis=1)
    w_al = jnp.stack([W_att_lang[:D_FEAT, 0], W_att_lang[D_FEAT:, 0]], axis=1)
    w1d = W1[0:2 * D_FEAT]
    w1s = jnp.concatenate([W1[2 * D_FEAT + D_EDGE + D_FEAT:],
                           W1[2 * D_FEAT + D_EDGE:2 * D_FEAT + D_EDGE + D_FEAT]],
                          axis=0)
    w1e = W1[2 * D_FEAT:2 * D_FEAT + D_EDGE]

    av, al, mv, ml = _tc_a(x, x_lang, w_av, w_al, W_msg, W_msg_lang,
                           interpret=interpret)

    if interpret:
        def _agg(a2, m):
            a_s, a_d = a2[:, 0], a2[:, 1]
            l = a_s[src] + a_d[dst]
            l = jnp.where(l >= 0, l, 0.2 * l)
            e = jnp.exp(l)
            s = jax.ops.segment_sum(e, dst, num_segments=N_NODES)
            scale = e / (s[dst] + 1e-16)
            return jax.ops.segment_sum(m[src] * scale[:, None], dst,
                                       num_segments=N_NODES)
        aggv = _agg(av, mv)
        aggl = _agg(al, ml)
    else:
        aggv, aggl = _sc_1(src, dst, av.reshape(2 * N_NODES),
                           al.reshape(2 * N_NODES), mv, ml)
        aggv = aggv[:N_NODES]
        aggl = aggl[:N_NODES]

    pd, ps = _tc_b(x, x_lang, aggv, aggl, w1d, w1s, interpret=interpret)

    if interpret:
        bd = pd[dst]
        bs = ps[src]
    else:
        bd, bs = _sc_2(src, dst, pd, ps)

    return _tc_c(bd, bs, edge_attr, w1e, b1.reshape(1, D_HID), W2,
                 b2.reshape(1, N_CLASSES), interpret=interpret)


def kernel(x, x_lang, edge_index, edge_attr, W_att, W_msg, W_att_lang,
           W_msg_lang, W1, b1, W2, b2):
    return _impl(x, x_lang, edge_index, edge_attr, W_att, W_msg, W_att_lang,
                 W_msg_lang, W1, b1, W2, b2, interpret=False)


# speedup vs baseline: 3.1772x; 1.3265x over previous
import functools

import jax
import jax.numpy as jnp
from jax import lax
from jax.experimental import pallas as pl
from jax.experimental.pallas import tpu as pltpu
from jax.experimental.pallas import tpu_sc as plsc

N_NODES = 10000
N_EDGES = 160000
D_FEAT = 128
D_EDGE = 16
D_HID = 256
N_CLASSES = 117

_NB = 1000
_EB = 2000


def _tca_body(x_ref, xl_ref, wav_ref, wal_ref, wm_ref, wml_ref,
              av_ref, al_ref, mv_ref, ml_ref):
    x = x_ref[...]
    xl = xl_ref[...]
    f32 = jnp.float32
    av_ref[...] = jax.lax.dot(x, wav_ref[...], preferred_element_type=f32)
    al_ref[...] = jax.lax.dot(xl, wal_ref[...], preferred_element_type=f32)
    mv_ref[...] = jax.lax.dot(x, wm_ref[...], preferred_element_type=f32)
    ml_ref[...] = jax.lax.dot(xl, wml_ref[...], preferred_element_type=f32)


def _tc_a(x, xl, w_av, w_al, wm, wml, interpret=False):
    nblk = N_NODES // _NB
    blk = lambda r: pl.BlockSpec((_NB, r), lambda i: (i, 0))
    full = lambda a, b: pl.BlockSpec((a, b), lambda i: (0, 0))
    return pl.pallas_call(
        _tca_body,
        grid=(nblk,),
        in_specs=[blk(D_FEAT), blk(D_FEAT), full(D_FEAT, 2), full(D_FEAT, 2),
                  full(D_FEAT, D_FEAT), full(D_FEAT, D_FEAT)],
        out_specs=[blk(2), blk(2), blk(D_FEAT), blk(D_FEAT)],
        out_shape=[jax.ShapeDtypeStruct((N_NODES, 2), jnp.float32),
                   jax.ShapeDtypeStruct((N_NODES, 2), jnp.float32),
                   jax.ShapeDtypeStruct((N_NODES, D_FEAT), jnp.float32),
                   jax.ShapeDtypeStruct((N_NODES, D_FEAT), jnp.float32)],
        interpret=interpret,
    )(x, xl, w_av, w_al, wm, wml)


def _tcb_body(x_ref, xl_ref, aggv_ref, aggl_ref, w1d_ref, w1s_ref,
              pd_ref, ps_ref):
    nv = jnp.maximum(x_ref[...] + aggv_ref[...], 0.0)
    nl = jnp.maximum(xl_ref[...] + aggl_ref[...], 0.0)
    cat = jnp.concatenate([nv, nl], axis=1)
    f32 = jnp.float32

    def pack(m):
        lo = lax.bitcast_convert_type(m[:, :D_FEAT], jnp.int32)
        hi = lax.bitcast_convert_type(m[:, D_FEAT:], jnp.int32)
        lo = lax.shift_right_logical(lo + 0x8000, 16)
        hi = (hi + 0x8000) & jnp.int32(-65536)
        return lo | hi

    pd_ref[...] = pack(jax.lax.dot(cat, w1d_ref[...],
                                   preferred_element_type=f32))
    ps_ref[...] = pack(jax.lax.dot(cat, w1s_ref[...],
                                   preferred_element_type=f32))


def _tc_b(x, xl, aggv, aggl, w1d, w1s, interpret=False):
    nblk = N_NODES // _NB
    blk = lambda r: pl.BlockSpec((_NB, r), lambda i: (i, 0))
    full = lambda a, b: pl.BlockSpec((a, b), lambda i: (0, 0))
    return pl.pallas_call(
        _tcb_body,
        grid=(nblk,),
        in_specs=[blk(D_FEAT), blk(D_FEAT), blk(D_FEAT), blk(D_FEAT),
                  full(2 * D_FEAT, D_HID), full(2 * D_FEAT, D_HID)],
        out_specs=[blk(D_HID // 2), blk(D_HID // 2)],
        out_shape=[jax.ShapeDtypeStruct((N_NODES, D_HID // 2), jnp.int32),
                   jax.ShapeDtypeStruct((N_NODES, D_HID // 2), jnp.int32)],
        interpret=interpret,
    )(x, xl, aggv, aggl, w1d, w1s)


def _tcc_body(bd_ref, bs_ref, ea_ref, w1e_ref, b1_ref, w2_ref, b2_ref,
              out_ref):
    f32 = jnp.float32
    e = jax.lax.dot(ea_ref[...], w1e_ref[...], preferred_element_type=f32)

    def unpack(w):
        lo = lax.bitcast_convert_type(lax.shift_left(w, 16), f32)
        hi = lax.bitcast_convert_type(w & jnp.int32(-65536), f32)
        return lo, hi

    dlo, dhi = unpack(bd_ref[...])
    slo, shi = unpack(bs_ref[...])
    base = jnp.concatenate([dlo + slo, dhi + shi], axis=1)
    h = jnp.maximum(base + e + b1_ref[...], 0.0)
    logit = jax.lax.dot(h, w2_ref[...], preferred_element_type=f32) + b2_ref[...]
    out_ref[...] = jax.nn.sigmoid(logit)


def _tc_c(bd, bs, ea, w1e, b1, w2, b2, interpret=False):
    nblk = N_EDGES // _EB
    blk = lambda r: pl.BlockSpec((_EB, r), lambda i: (i, 0))
    full = lambda a, b: pl.BlockSpec((a, b), lambda i: (0, 0))
    return pl.pallas_call(
        _tcc_body,
        grid=(nblk,),
        in_specs=[blk(D_HID // 2), blk(D_HID // 2), blk(D_EDGE),
                  full(D_EDGE, D_HID), full(1, D_HID),
                  full(D_HID, N_CLASSES), full(1, N_CLASSES)],
        out_specs=blk(N_CLASSES),
        out_shape=jax.ShapeDtypeStruct((N_EDGES, N_CLASSES), jnp.float32),
        interpret=interpret,
    )(bd, bs, ea, w1e, b1, w2, b2)


_SC1_EPW = N_EDGES // 16
_SC1_K = 80
_SROWS = 80
_NPAD = 10048
_NPW = 632


def _sc1_body(src_hbm, dst_hbm, av_hbm, al_hbm, mv_hbm, ml_hbm,
              aggv_hbm, aggl_hbm,
              a_tab, s_part, sc1_c, dc1_c, idx_g, idx_c, rows_v, sh_all, sem):
    c = lax.axis_index("c")
    t = lax.axis_index("s")
    ebase = t * _SC1_EPW
    iota16 = lax.iota(jnp.int32, 16)

    if True:
        def zs(i, _):
            for cc in range(8):
                s_part[i, pl.ds(cc * 16, 16)] = jnp.zeros((16,), jnp.float32)
                rows_v[i, pl.ds(cc * 16, 16)] = jnp.zeros((16,), jnp.float32)
            return ()
        lax.fori_loop(0, _SROWS, zs, ())

        for i in range(5):
            idx_g[pl.ds(i * 16, 16)] = iota16 + (_NPAD + i * 16)

        @pl.when(t == 0)
        def _():
            pltpu.sync_copy(rows_v, sh_all.at[pl.ds(_NPAD, _SROWS)])
        zbase = pl.multiple_of(t * _NPW, 8)
        for q in range(7):
            pltpu.sync_copy(rows_v, sh_all.at[pl.ds(zbase + q * 80, 80)])

        @pl.when(t < 15)
        def _():
            pltpu.sync_copy(rows_v.at[pl.ds(0, 72)],
                            sh_all.at[pl.ds(zbase + 560, 72)])

        @pl.when(t == 15)
        def _():
            pltpu.sync_copy(rows_v.at[pl.ds(0, 8)],
                            sh_all.at[pl.ds(15 * _NPW + 560, 8)])

        @pl.when(c == 0)
        def _():
            pltpu.sync_copy(av_hbm, a_tab)

        @pl.when(c == 1)
        def _():
            pltpu.sync_copy(al_hbm, a_tab)

        plsc.subcore_barrier()

        def p1c(k, _):
            pltpu.sync_copy(src_hbm.at[pl.ds(ebase + k * 2000, 2000)], sc1_c)
            pltpu.sync_copy(dst_hbm.at[pl.ds(ebase + k * 2000, 2000)], dc1_c)

            def p1(i, _):
                sv = sc1_c[pl.ds(i * 16, 16)]
                dv = dc1_c[pl.ds(i * 16, 16)]
                asg = plsc.load_gather(a_tab, [sv * 2])
                adg = plsc.load_gather(a_tab, [dv * 2 + 1])
                l = asg + adg
                l = jnp.where(l >= 0.0, l, l * 0.2)
                e = jnp.exp(l)
                plsc.addupdate_scatter(
                    s_part, [lax.shift_right_logical(dv, 7), dv & 127], e)
                return ()
            lax.fori_loop(0, 125, p1, ())
            return ()
        lax.fori_loop(0, _SC1_EPW // 2000, p1c, ())

        pltpu.sync_copy(s_part, sh_all.at[idx_g], add=True)
        plsc.subcore_barrier()
        pltpu.sync_copy(sh_all.at[pl.ds(_NPAD, _SROWS)], s_part)

        def p2(j, _):
            pltpu.sync_copy(src_hbm.at[pl.ds(ebase + j * _SC1_K, _SC1_K)], idx_g)
            pltpu.sync_copy(dst_hbm.at[pl.ds(ebase + j * _SC1_K, _SC1_K)], idx_c)

            @pl.when(c == 0)
            def _():
                pltpu.async_copy(mv_hbm.at[idx_g], rows_v, sem).wait()

            @pl.when(c == 1)
            def _():
                pltpu.async_copy(ml_hbm.at[idx_g], rows_v, sem).wait()

            def scale_group(g, _):
                sv = idx_g[pl.ds(g * 16, 16)]
                dv = idx_c[pl.ds(g * 16, 16)]
                asg = plsc.load_gather(a_tab, [sv * 2])
                adg = plsc.load_gather(a_tab, [dv * 2 + 1])
                l = asg + adg
                l = jnp.where(l >= 0.0, l, l * 0.2)
                e = jnp.exp(l)
                sg = plsc.load_gather(
                    s_part, [lax.shift_right_logical(dv, 7), dv & 127])
                scale = e / (sg + 1e-16)
                for r in range(16):
                    sc = scale[r]
                    row = g * 16 + r
                    for cc in range(8):
                        rows_v[row, pl.ds(cc * 16, 16)] = (
                            rows_v[row, pl.ds(cc * 16, 16)] * sc)
                return ()
            lax.fori_loop(0, _SC1_K // 16, scale_group, ())

            pltpu.sync_copy(rows_v, sh_all.at[idx_c], add=True)
            return ()
        lax.fori_loop(0, _SC1_EPW // _SC1_K, p2, ())

        plsc.subcore_barrier()

        @pl.when((c == 0) & (t < 15))
        def _():
            pltpu.sync_copy(sh_all.at[pl.ds(zbase, _NPW)],
                            aggv_hbm.at[pl.ds(zbase, _NPW)])

        @pl.when((c == 0) & (t == 15))
        def _():
            pltpu.sync_copy(sh_all.at[pl.ds(15 * _NPW, _NPAD - 15 * _NPW)],
                            aggv_hbm.at[pl.ds(15 * _NPW, _NPAD - 15 * _NPW)])

        @pl.when((c == 1) & (t < 15))
        def _():
            pltpu.sync_copy(sh_all.at[pl.ds(zbase, _NPW)],
                            aggl_hbm.at[pl.ds(zbase, _NPW)])

        @pl.when((c == 1) & (t == 15))
        def _():
            pltpu.sync_copy(sh_all.at[pl.ds(15 * _NPW, _NPAD - 15 * _NPW)],
                            aggl_hbm.at[pl.ds(15 * _NPW, _NPAD - 15 * _NPW)])


def _sc_1(src, dst, av_flat, al_flat, mv, ml):
    f = pl.kernel(
        _sc1_body,
        mesh=plsc.VectorSubcoreMesh(core_axis_name="c", subcore_axis_name="s"),
        compiler_params=pltpu.CompilerParams(needs_layout_passes=False),
        out_type=[jax.ShapeDtypeStruct((_NPAD, D_FEAT), jnp.float32),
                  jax.ShapeDtypeStruct((_NPAD, D_FEAT), jnp.float32)],
        scratch_types=[pltpu.VMEM((2 * N_NODES,), jnp.float32),
                       pltpu.VMEM((_SROWS, 128), jnp.float32),
                       pltpu.VMEM((2000,), jnp.int32),
                       pltpu.VMEM((2000,), jnp.int32),
                       pltpu.VMEM((_SC1_K,), jnp.int32),
                       pltpu.VMEM((_SC1_K,), jnp.int32),
                       pltpu.VMEM((_SC1_K, D_FEAT), jnp.float32),
                       pltpu.VMEM_SHARED((_NPAD + _SROWS, D_FEAT), jnp.float32),
                       pltpu.SemaphoreType.DMA],
    )
    return f(src, dst, av_flat, al_flat, mv, ml)


_SC2_EPW = N_EDGES // 32
_SC2_K = 128
_SC2_NCH = 40


def _sc2_body(src_hbm, dst_hbm, pd_hbm, ps_hbm, outd_hbm, outs_hbm,
              idx_d0, idx_d1, idx_s0, idx_s1, rd0, rd1, rs0, rs1,
              sem_gd, sem_gs, sem_wd, sem_ws):
    c = lax.axis_index("c")
    s = lax.axis_index("s")
    wid = s * 2 + c
    base = wid * _SC2_EPW
    idx_d = [idx_d0, idx_d1]
    idx_s = [idx_s0, idx_s1]
    rd = [rd0, rd1]
    rs = [rs0, rs1]

    def off(j):
        return base + min(j * _SC2_K, _SC2_EPW - _SC2_K)

    def load_idx(j):
        b = j % 2
        pltpu.sync_copy(dst_hbm.at[pl.ds(off(j), _SC2_K)], idx_d[b])
        pltpu.sync_copy(src_hbm.at[pl.ds(off(j), _SC2_K)], idx_s[b])

    def fire_gathers(j):
        b = j % 2
        return (pltpu.async_copy(pd_hbm.at[idx_d[b]], rd[b], sem_gd),
                pltpu.async_copy(ps_hbm.at[idx_s[b]], rs[b], sem_gs))

    def fire_writes(j):
        b = j % 2
        return (pltpu.async_copy(rd[b], outd_hbm.at[pl.ds(off(j), _SC2_K)],
                                 sem_wd),
                pltpu.async_copy(rs[b], outs_hbm.at[pl.ds(off(j), _SC2_K)],
                                 sem_ws))

    load_idx(0)
    g = fire_gathers(0)
    w_prev = None
    for j in range(_SC2_NCH):
        if j + 1 < _SC2_NCH:
            load_idx(j + 1)
        g[0].wait()
        g[1].wait()
        if j + 1 < _SC2_NCH:
            if w_prev is not None:
                w_prev[0].wait()
                w_prev[1].wait()
            g = fire_gathers(j + 1)
        w = fire_writes(j)
        w_prev, w = w, None
    w_prev[0].wait()
    w_prev[1].wait()


def _sc_2(src, dst, pd, ps):
    f = pl.kernel(
        _sc2_body,
        mesh=plsc.VectorSubcoreMesh(core_axis_name="c", subcore_axis_name="s"),
        compiler_params=pltpu.CompilerParams(needs_layout_passes=False),
        out_type=[jax.ShapeDtypeStruct((N_EDGES, D_HID // 2), jnp.int32),
                  jax.ShapeDtypeStruct((N_EDGES, D_HID // 2), jnp.int32)],
        scratch_types=[pltpu.VMEM((_SC2_K,), jnp.int32),
                       pltpu.VMEM((_SC2_K,), jnp.int32),
                       pltpu.VMEM((_SC2_K,), jnp.int32),
                       pltpu.VMEM((_SC2_K,), jnp.int32),
                       pltpu.VMEM((_SC2_K, D_HID // 2), jnp.int32),
                       pltpu.VMEM((_SC2_K, D_HID // 2), jnp.int32),
                       pltpu.VMEM((_SC2_K, D_HID // 2), jnp.int32),
                       pltpu.VMEM((_SC2_K, D_HID // 2), jnp.int32),
                       pltpu.SemaphoreType.DMA,
                       pltpu.SemaphoreType.DMA,
                       pltpu.SemaphoreType.DMA,
                       pltpu.SemaphoreType.DMA],
    )
    return f(src, dst, pd, ps)


def _impl(x, x_lang, edge_index, edge_attr, W_att, W_msg, W_att_lang,
          W_msg_lang, W1, b1, W2, b2, interpret=False):
    src = edge_index[0].astype(jnp.int32)
    dst = edge_index[1].astype(jnp.int32)
    w_av = jnp.stack([W_att[:D_FEAT, 0], W_att[D_FEAT:, 0]], axis=1)
    w_al = jnp.stack([W_att_lang[:D_FEAT, 0], W_att_lang[D_FEAT:, 0]], axis=1)
    w1d = W1[0:2 * D_FEAT]
    w1s = jnp.concatenate([W1[2 * D_FEAT + D_EDGE + D_FEAT:],
                           W1[2 * D_FEAT + D_EDGE:2 * D_FEAT + D_EDGE + D_FEAT]],
                          axis=0)
    w1e = W1[2 * D_FEAT:2 * D_FEAT + D_EDGE]

    av, al, mv, ml = _tc_a(x, x_lang, w_av, w_al, W_msg, W_msg_lang,
                           interpret=interpret)

    if interpret:
        def _agg(a2, m):
            a_s, a_d = a2[:, 0], a2[:, 1]
            l = a_s[src] + a_d[dst]
            l = jnp.where(l >= 0, l, 0.2 * l)
            e = jnp.exp(l)
            s = jax.ops.segment_sum(e, dst, num_segments=N_NODES)
            scale = e / (s[dst] + 1e-16)
            return jax.ops.segment_sum(m[src] * scale[:, None], dst,
                                       num_segments=N_NODES)
        aggv = _agg(av, mv)
        aggl = _agg(al, ml)
    else:
        aggv, aggl = _sc_1(src, dst, av.reshape(2 * N_NODES),
                           al.reshape(2 * N_NODES), mv, ml)
        aggv = aggv[:N_NODES]
        aggl = aggl[:N_NODES]

    pd, ps = _tc_b(x, x_lang, aggv, aggl, w1d, w1s, interpret=interpret)

    if interpret:
        bd = pd[dst]
        bs = ps[src]
    else:
        bd, bs = _sc_2(src, dst, pd, ps)

    return _tc_c(bd, bs, edge_attr, w1e, b1.reshape(1, D_HID), W2,
                 b2.reshape(1, N_CLASSES), interpret=interpret)


def kernel(x, x_lang, edge_index, edge_attr, W_att, W_msg, W_att_lang,
           W_msg_lang, W1, b1, W2, b2):
    return _impl(x, x_lang, edge_index, edge_attr, W_att, W_msg, W_att_lang,
                 W_msg_lang, W1, b1, W2, b2, interpret=False)
